# Initial kernel scaffold; baseline (speedup 1.0000x reference)
import jax
import jax.numpy as jnp
from jax.experimental import pallas as pl


def kernel(output_rv, output_mask_logits, target):
    raise NotImplementedError("write your pallas kernel here")


# speedup vs baseline: 5.0048x; 5.0048x over previous
import functools

import jax
import jax.numpy as jnp
import numpy as np
from jax.experimental import pallas as pl

B, T, H, W = 2, 2, 64, 256
FOV_UP = 3.0 * np.pi / 180.0
FOV_DOWN = -25.0 * np.pi / 180.0
MASK_THRESHOLD = 0.5
BT = B * T
N = H * W + 1
NPAD = 16896
TQ = 528
TR = 4224
NQT = NPAD // TQ
NRT = NPAD // TR
NPROB = 2 * BT


def _chamfer_body(qn_ref, rt_ref, s_ref, c_ref):
    q = pl.program_id(1)
    qx = qn_ref[0, :, 0:1]
    qy = qn_ref[0, :, 1:2]
    qz = qn_ref[0, :, 2:3]
    m = jnp.full((TQ, 1), jnp.inf, jnp.float32)
    for t in range(NRT):
        sl = pl.ds(t * TR, TR)
        rx = rt_ref[0, 0:1, sl]
        ry = rt_ref[0, 1:2, sl]
        rz = rt_ref[0, 2:3, sl]
        d = (qx - rx) ** 2 + (qy - ry) ** 2 + (qz - rz) ** 2
        m = jnp.minimum(m, jnp.min(d, axis=1, keepdims=True))
    s = jnp.sum(m)
    c = jnp.sum((m > 0.0).astype(jnp.float32))
    sv = jnp.full((1, 1, 128), s, jnp.float32)
    cv = jnp.full((1, 1, 128), c, jnp.float32)

    @pl.when(q == 0)
    def _():
        s_ref[...] = sv
        c_ref[...] = cv

    @pl.when(q != 0)
    def _():
        s_ref[...] = s_ref[...] + sv
        c_ref[...] = c_ref[...] + cv


@functools.partial(jax.jit)
def _chamfer(output_rv, output_mask_logits, target):
    mask_prob = jax.nn.sigmoid(output_mask_logits)
    masked_rv = jnp.where(mask_prob > MASK_THRESHOLD, output_rv, -1.0)
    rv = masked_rv.reshape(BT, H, W)

    h = jnp.arange(H, dtype=jnp.float32)
    w = jnp.arange(W, dtype=jnp.float32)
    yaw = -((w + 0.5) / W * 2.0 - 1.0) * jnp.pi
    pitch = (1.0 - (h + 0.5) / H) * (FOV_UP - FOV_DOWN) + FOV_DOWN
    yaw2 = jnp.broadcast_to(yaw[None, :], (H, W))
    pitch2 = jnp.broadcast_to(pitch[:, None], (H, W))
    x = rv * (jnp.cos(pitch2) * jnp.cos(yaw2))[None]
    y = rv * (jnp.cos(pitch2) * jnp.sin(yaw2))[None]
    z = rv * jnp.sin(pitch2)[None]
    valid = rv > 0.0
    ox = jnp.where(valid, x, 1000.0).reshape(BT, H * W)
    oy = jnp.where(valid, y, 1000.0).reshape(BT, H * W)
    oz = jnp.where(valid, z, 1000.0).reshape(BT, H * W)
    out_pts = jnp.stack([ox, oy, oz], axis=-1)

    tvalid = (target[:, :, 0] >= 0.0).reshape(BT, H, W)
    txyz = target[:, :, 1:4].reshape(BT, 3, H, W)
    t_pts = jnp.where(tvalid[:, None], txyz, 1000.0)
    t_pts = t_pts.reshape(BT, 3, H * W).transpose(0, 2, 1)

    def pad_pts(p):
        p = jnp.pad(p, ((0, 0), (0, NPAD - H * W), (0, 0)),
                    constant_values=1000.0)
        return jnp.pad(p, ((0, 0), (0, 0), (0, 5)))

    out_p = pad_pts(out_pts)
    tgt_p = pad_pts(t_pts)

    qn = jnp.concatenate([out_p, tgt_p], axis=0)
    rt = jnp.concatenate([tgt_p, out_p], axis=0).transpose(0, 2, 1)

    s, c = pl.pallas_call(
        _chamfer_body,
        grid=(NPROB, NQT),
        in_specs=[
            pl.BlockSpec((1, TQ, 8), lambda p, q: (p, q, 0)),
            pl.BlockSpec((1, 8, NPAD), lambda p, q: (p, 0, 0)),
        ],
        out_specs=[
            pl.BlockSpec((1, 1, 128), lambda p, q: (p, 0, 0)),
            pl.BlockSpec((1, 1, 128), lambda p, q: (p, 0, 0)),
        ],
        out_shape=[
            jax.ShapeDtypeStruct((NPROB, 1, 128), jnp.float32),
            jax.ShapeDtypeStruct((NPROB, 1, 128), jnp.float32),
        ],
    )(qn, rt)
    s = s[:, 0, 0]
    c = c[:, 0, 0]

    dist_combined = s[:BT] / c[:BT] + s[BT:] / c[BT:]
    chamfer_distances_tensor = dist_combined.reshape(T, B)
    chamf_dist_t = jnp.mean(chamfer_distances_tensor, axis=1)
    return chamf_dist_t, chamfer_distances_tensor


def kernel(output_rv, output_mask_logits, target):
    return _chamfer(output_rv, output_mask_logits, target)


# speedup vs baseline: 10.5620x; 2.1104x over previous
import functools

import jax
import jax.numpy as jnp
import numpy as np
from jax.experimental import pallas as pl

B, T, H, W = 2, 2, 64, 256
FOV_UP = 3.0 * np.pi / 180.0
FOV_DOWN = -25.0 * np.pi / 180.0
MASK_THRESHOLD = 0.5
BT = B * T
N = H * W + 1
NPAD = 16896
TQ = 512
TR = 4224
NQT = NPAD // TQ
NRT = NPAD // TR
NPROB = 2 * BT


def _chamfer_body(qn_ref, rt_ref, s_ref, c_ref):
    q = pl.program_id(1)
    qb = qn_ref[0]
    na = jnp.sum(qb * qb, axis=1, keepdims=True)
    m = jnp.full((TQ, 1), jnp.inf, jnp.float32)
    for t in range(NRT):
        rsl = rt_ref[0, :, pl.ds(t * TR, TR)]
        nb = jnp.sum(rsl * rsl, axis=0, keepdims=True)
        d2 = jax.lax.dot_general(qb, rsl, (((1,), (0,)), ((), ())),
                                 preferred_element_type=jnp.float32)
        v = nb - 2.0 * d2
        m = jnp.minimum(m, jnp.min(v, axis=1, keepdims=True))
    dist = na + m
    s = jnp.sum(dist)
    c = jnp.sum((dist > 0.0).astype(jnp.float32))
    sv = jnp.full((1, 1, 128), s, jnp.float32)
    cv = jnp.full((1, 1, 128), c, jnp.float32)

    @pl.when(q == 0)
    def _():
        s_ref[...] = sv
        c_ref[...] = cv

    @pl.when(q != 0)
    def _():
        s_ref[...] = s_ref[...] + sv
        c_ref[...] = c_ref[...] + cv


@functools.partial(jax.jit)
def _chamfer(output_rv, output_mask_logits, target):
    mask_prob = jax.nn.sigmoid(output_mask_logits)
    masked_rv = jnp.where(mask_prob > MASK_THRESHOLD, output_rv, -1.0)
    rv = masked_rv.reshape(BT, H, W)

    h = jnp.arange(H, dtype=jnp.float32)
    w = jnp.arange(W, dtype=jnp.float32)
    yaw = -((w + 0.5) / W * 2.0 - 1.0) * jnp.pi
    pitch = (1.0 - (h + 0.5) / H) * (FOV_UP - FOV_DOWN) + FOV_DOWN
    yaw2 = jnp.broadcast_to(yaw[None, :], (H, W))
    pitch2 = jnp.broadcast_to(pitch[:, None], (H, W))
    x = rv * (jnp.cos(pitch2) * jnp.cos(yaw2))[None]
    y = rv * (jnp.cos(pitch2) * jnp.sin(yaw2))[None]
    z = rv * jnp.sin(pitch2)[None]
    valid = rv > 0.0
    ox = jnp.where(valid, x, 1000.0).reshape(BT, H * W)
    oy = jnp.where(valid, y, 1000.0).reshape(BT, H * W)
    oz = jnp.where(valid, z, 1000.0).reshape(BT, H * W)
    out_pts = jnp.stack([ox, oy, oz], axis=-1)

    tvalid = (target[:, :, 0] >= 0.0).reshape(BT, H, W)
    txyz = target[:, :, 1:4].reshape(BT, 3, H, W)
    t_pts = jnp.where(tvalid[:, None], txyz, 1000.0)
    t_pts = t_pts.reshape(BT, 3, H * W).transpose(0, 2, 1)

    def pad_pts(p):
        p = jnp.pad(p, ((0, 0), (0, NPAD - H * W), (0, 0)),
                    constant_values=1000.0)
        return jnp.pad(p, ((0, 0), (0, 0), (0, 5)))

    out_p = pad_pts(out_pts)
    tgt_p = pad_pts(t_pts)

    qn = jnp.concatenate([out_p, tgt_p], axis=0)
    rt = jnp.concatenate([tgt_p, out_p], axis=0).transpose(0, 2, 1)

    s, c = pl.pallas_call(
        _chamfer_body,
        grid=(NPROB, NQT),
        in_specs=[
            pl.BlockSpec((1, TQ, 8), lambda p, q: (p, q, 0)),
            pl.BlockSpec((1, 8, NPAD), lambda p, q: (p, 0, 0)),
        ],
        out_specs=[
            pl.BlockSpec((1, 1, 128), lambda p, q: (p, 0, 0)),
            pl.BlockSpec((1, 1, 128), lambda p, q: (p, 0, 0)),
        ],
        out_shape=[
            jax.ShapeDtypeStruct((NPROB, 1, 128), jnp.float32),
            jax.ShapeDtypeStruct((NPROB, 1, 128), jnp.float32),
        ],
    )(qn, rt)
    s = s[:, 0, 0]
    c = c[:, 0, 0]

    dist_combined = s[:BT] / c[:BT] + s[BT:] / c[BT:]
    chamfer_distances_tensor = dist_combined.reshape(T, B)
    chamf_dist_t = jnp.mean(chamfer_distances_tensor, axis=1)
    return chamf_dist_t, chamfer_distances_tensor


def kernel(output_rv, output_mask_logits, target):
    return _chamfer(output_rv, output_mask_logits, target)
